# baseline (speedup 1.0000x reference)
import functools

import jax
import jax.numpy as jnp
from jax import lax
from jax.experimental import pallas as pl
from jax.experimental.pallas import tpu as pltpu
from jax.experimental.pallas import tpu_sc as plsc

V, D, B, N_NEG = 100000, 64, 16384, 5
NC, NS, L = 2, 16, 16
NW = NC * NS
BPW = B // NW
SUB = 128
NSUB = BPW // SUB
NGRP = SUB // L
NSCORE = 1 + N_NEG
NROW = 2 + N_NEG


def _sc_scores(center, context, neg_t, center_w, context_w):
    mesh = plsc.VectorSubcoreMesh(core_axis_name="c", subcore_axis_name="s")

    @functools.partial(
        pl.kernel,
        out_type=jax.ShapeDtypeStruct((NW * NSCORE, BPW), jnp.float32),
        mesh=mesh,
        compiler_params=pltpu.CompilerParams(
            needs_layout_passes=False, use_tc_tiling_on_sc=False),
        scratch_types=[
            [[pltpu.VMEM((SUB,), jnp.int32) for _ in range(NROW)]
             for _ in range(2)],
            [[pltpu.VMEM((SUB, D), jnp.float32) for _ in range(NROW)]
             for _ in range(2)],
            pltpu.VMEM((NSCORE, BPW), jnp.float32),
            [pltpu.SemaphoreType.DMA for _ in range(2)],
        ],
    )
    def scores_kernel(center_h, context_h, negt_h, cw_h, xw_h, out_h,
                      idx, rows_buf, scores, sem):
        wid = lax.axis_index("s") * NC + lax.axis_index("c")
        base = wid * BPW

        def issue(c, slot):
            boff = base + c * SUB
            srcs = [center_h.at[pl.ds(boff, SUB)],
                    context_h.at[pl.ds(boff, SUB)]]
            for j in range(N_NEG):
                srcs.append(negt_h.at[pl.ds(j * B + boff, SUB)])
            copies = [pltpu.async_copy(s, idx[slot][r], sem[slot])
                      for r, s in enumerate(srcs)]
            for cp in copies:
                cp.wait()
            tables = [cw_h] + [xw_h] * (NROW - 1)
            return [pltpu.async_copy(t.at[idx[slot][r]], rows_buf[slot][r],
                                     sem[slot])
                    for r, t in enumerate(tables)]

        def compute(c, slot):
            c_rows, x_rows = rows_buf[slot][0], rows_buf[slot][1]
            n_rows = rows_buf[slot][2:]

            def group_body(g, carry):
                rows = g * L + lax.iota(jnp.int32, L)

                def d_body(d, accs):
                    col = jnp.full((L,), d, jnp.int32)
                    cv = plsc.load_gather(c_rows, [rows, col])
                    xv = plsc.load_gather(x_rows, [rows, col])
                    new = [accs[0] + cv * xv]
                    for j in range(N_NEG):
                        nv = plsc.load_gather(n_rows[j], [rows, col])
                        new.append(accs[1 + j] + cv * nv)
                    return tuple(new)

                z = jnp.zeros((L,), jnp.float32)
                accs = lax.fori_loop(0, D, d_body, (z,) * NSCORE, unroll=1)
                off = c * SUB + g * L
                scores[0, pl.ds(off, L)] = accs[0]
                for j in range(N_NEG):
                    scores[1 + j, pl.ds(off, L)] = -accs[1 + j]
                return carry

            lax.fori_loop(0, NGRP, group_body, 0)

        handles = {0: issue(0, 0), 1: issue(1, 1)}
        for c in range(NSUB):
            slot = c % 2
            for cp in handles.pop(c):
                cp.wait()
            compute(c, slot)
            if c + 2 < NSUB:
                handles[c + 2] = issue(c + 2, slot)
        pltpu.sync_copy(scores, out_h.at[pl.ds(wid * NSCORE, NSCORE)])

    return scores_kernel(center, context, neg_t, center_w, context_w)


def _tc_loss(scores2d):
    def body(s_ref, o_ref):
        o_ref[0, 0] = -jnp.sum(jax.nn.log_sigmoid(s_ref[...])) / B

    out = pl.pallas_call(
        body,
        out_shape=jax.ShapeDtypeStruct((1, 1), jnp.float32),
        out_specs=pl.BlockSpec(memory_space=pltpu.SMEM),
    )(scores2d)
    return out[0, 0]


def kernel(center, context, negatives, center_w, context_w):
    center = center.astype(jnp.int32)
    context = context.astype(jnp.int32)
    neg_t = negatives.astype(jnp.int32).T.reshape(-1)
    scores = _sc_scores(center, context, neg_t, center_w, context_w)
    return _tc_loss(scores)


# speedup vs baseline: 1.0872x; 1.0872x over previous
import functools

import jax
import jax.numpy as jnp
from jax import lax
from jax.experimental import pallas as pl
from jax.experimental.pallas import tpu as pltpu
from jax.experimental.pallas import tpu_sc as plsc

V, D, B, N_NEG = 100000, 64, 16384, 5
NC, NS, L = 2, 16, 16
NW = NC * NS
BPW = B // NW
SUB = 128
NSUB = BPW // SUB
NGRP = SUB // L
NSCORE = 1 + N_NEG
NROW = 2 + N_NEG


def _sc_scores(center, context, neg_t, center_w, context_w):
    mesh = plsc.VectorSubcoreMesh(core_axis_name="c", subcore_axis_name="s")

    @functools.partial(
        pl.kernel,
        out_type=jax.ShapeDtypeStruct((NW * NSCORE, BPW), jnp.float32),
        mesh=mesh,
        compiler_params=pltpu.CompilerParams(
            needs_layout_passes=False, use_tc_tiling_on_sc=False),
        scratch_types=[
            [[pltpu.VMEM((SUB,), jnp.int32) for _ in range(NROW)]
             for _ in range(2)],
            [[pltpu.VMEM((SUB, D), jnp.float32) for _ in range(NROW)]
             for _ in range(2)],
            pltpu.VMEM((NSCORE, BPW), jnp.float32),
            [pltpu.SemaphoreType.DMA for _ in range(2)],
        ],
    )
    def scores_kernel(center_h, context_h, negt_h, cw_h, xw_h, out_h,
                      idx, rows_buf, scores, sem):
        wid = lax.axis_index("s") * NC + lax.axis_index("c")
        base = wid * BPW

        def issue(c, slot):
            boff = base + c * SUB
            srcs = [center_h.at[pl.ds(boff, SUB)],
                    context_h.at[pl.ds(boff, SUB)]]
            for j in range(N_NEG):
                srcs.append(negt_h.at[pl.ds(j * B + boff, SUB)])
            copies = [pltpu.async_copy(s, idx[slot][r], sem[slot])
                      for r, s in enumerate(srcs)]
            for cp in copies:
                cp.wait()
            tables = [cw_h] + [xw_h] * (NROW - 1)
            return [pltpu.async_copy(t.at[idx[slot][r]], rows_buf[slot][r],
                                     sem[slot])
                    for r, t in enumerate(tables)]

        def compute(c, slot):
            c_rows, x_rows = rows_buf[slot][0], rows_buf[slot][1]
            n_rows = rows_buf[slot][2:]

            def group_body(g, carry):
                rows = g * L + lax.iota(jnp.int32, L)

                def d_body(d, accs):
                    col = jnp.full((L,), d, jnp.int32)
                    cv = plsc.load_gather(c_rows, [rows, col])
                    xv = plsc.load_gather(x_rows, [rows, col])
                    new = [accs[0] + cv * xv]
                    for j in range(N_NEG):
                        nv = plsc.load_gather(n_rows[j], [rows, col])
                        new.append(accs[1 + j] + cv * nv)
                    return tuple(new)

                z = jnp.zeros((L,), jnp.float32)
                accs = lax.fori_loop(0, D, d_body, (z,) * NSCORE, unroll=2)
                off = c * SUB + g * L
                scores[0, pl.ds(off, L)] = accs[0]
                for j in range(N_NEG):
                    scores[1 + j, pl.ds(off, L)] = -accs[1 + j]
                return carry

            lax.fori_loop(0, NGRP, group_body, 0)

        handles = {0: issue(0, 0), 1: issue(1, 1)}
        for c in range(NSUB):
            slot = c % 2
            for cp in handles.pop(c):
                cp.wait()
            compute(c, slot)
            if c + 2 < NSUB:
                handles[c + 2] = issue(c + 2, slot)
        pltpu.sync_copy(scores, out_h.at[pl.ds(wid * NSCORE, NSCORE)])

    return scores_kernel(center, context, neg_t, center_w, context_w)


def _tc_loss(scores2d):
    def body(s_ref, o_ref):
        o_ref[0, 0] = -jnp.sum(jax.nn.log_sigmoid(s_ref[...])) / B

    out = pl.pallas_call(
        body,
        out_shape=jax.ShapeDtypeStruct((1, 1), jnp.float32),
        out_specs=pl.BlockSpec(memory_space=pltpu.SMEM),
    )(scores2d)
    return out[0, 0]


def kernel(center, context, negatives, center_w, context_w):
    center = center.astype(jnp.int32)
    context = context.astype(jnp.int32)
    neg_t = negatives.astype(jnp.int32).T.reshape(-1)
    scores = _sc_scores(center, context, neg_t, center_w, context_w)
    return _tc_loss(scores)
